# baseline (speedup 1.0000x reference)
import functools

import jax
import jax.numpy as jnp
from jax import lax
from jax.experimental import pallas as pl
from jax.experimental.pallas import tpu as pltpu, tpu_sc as plsc

L = 16


def _tc_k_update(pos_ref, kin, kval, kout, *, bh_blk, s):
    kout[...] = kin[...]
    for t in range(bh_blk):
        for j in range(s):
            p = pos_ref[j]
            kout[t, pl.ds(p, 1), :] = kval[t, pl.ds(j, 1), :]


def _sc_v_update(pos_hbm, vval_hbm, vcache_hbm, vout_hbm,
                 pos_v, eff_v, idx_out_v, idx_val_v, vrows_v,
                 stage_v, sem_ld, sem_st, sem_g, sem_s,
                 *, nw, rows_per, bh_per, s_max, s):
    wid = lax.axis_index("c") * (nw // 2) + lax.axis_index("s")
    base = wid * rows_per

    ch = stage_v[0].shape[0]
    n_ch = rows_per // ch
    works = [(vcache_hbm, vout_hbm, base + i * ch) for i in range(n_ch)]
    nbuf = len(stage_v)
    ld_d = [None] * len(works)
    st_d = [None] * len(works)

    for i in range(len(works)):
        b = i % nbuf
        if i >= nbuf:
            st_d[i - nbuf].wait()
        src, dst, off = works[i]
        ld_d[i] = pltpu.async_copy(src.at[pl.ds(off, ch)], stage_v[b],
                                   sem_ld[b])
        j = i - 2
        if j >= 0:
            ld_d[j].wait()
            srcj, dstj, offj = works[j]
            st_d[j] = pltpu.async_copy(stage_v[j % nbuf],
                                       dstj.at[pl.ds(offj, ch)],
                                       sem_st[j % nbuf])
    for j in (len(works) - 2, len(works) - 1):
        ld_d[j].wait()
        srcj, dstj, offj = works[j]
        st_d[j] = pltpu.async_copy(stage_v[j % nbuf],
                                   dstj.at[pl.ds(offj, ch)], sem_st[j % nbuf])

    pltpu.sync_copy(pos_hbm, pos_v.at[pl.ds(0, s)])
    pos_v[pl.ds(s, L)] = jnp.full((L,), -1, jnp.int32)
    pos0 = pos_v[pl.ds(0, L)]
    pos1 = pos_v[pl.ds(L, L)]
    nxt0 = pos_v[pl.ds(1, L)]
    nxt1 = pos_v[pl.ds(L + 1, L)]

    j0 = lax.iota(jnp.int32, L)
    big = jnp.int32(1 << 20)
    eff_v[pl.ds(0, L)] = jnp.where(pos0 != nxt0, j0, big)
    eff_v[pl.ds(L, L)] = jnp.where(pos1 != nxt1, j0 + L, big)
    eff_v[pl.ds(2 * L, L)] = jnp.full((L,), big, jnp.int32)
    k = 1
    while k < 2 * L:
        n0 = jnp.minimum(eff_v[pl.ds(0, L)], eff_v[pl.ds(k, L)])
        n1 = jnp.minimum(eff_v[pl.ds(L, L)], eff_v[pl.ds(L + k, L)])
        eff_v[pl.ds(0, L)] = n0
        eff_v[pl.ds(L, L)] = n1
        k *= 2
    eff0 = eff_v[pl.ds(0, L)]
    eff1 = eff_v[pl.ds(L, L)]

    for t in range(bh_per):
        bh = wid * bh_per + t
        idx_out_v[pl.ds(t * s, L)] = pos0 + bh * s_max
        idx_out_v[pl.ds(t * s + L, L)] = pos1 + bh * s_max
        idx_val_v[pl.ds(t * s, L)] = eff0 + bh * s
        idx_val_v[pl.ds(t * s + L, L)] = eff1 + bh * s

    g_v = pltpu.async_copy(vval_hbm.at[idx_val_v], vrows_v, sem_g)
    g_v.wait()
    for d_ in st_d[-nbuf:]:
        d_.wait()
    s_v = pltpu.async_copy(vrows_v, vout_hbm.at[idx_out_v], sem_s)
    s_v.wait()


def kernel(input_pos, k_val, v_val, k_cache, v_cache):
    b, h, s_max, d = k_cache.shape
    s = k_val.shape[2]
    bh = b * h
    total_rows = bh * s_max

    mesh = plsc.VectorSubcoreMesh(core_axis_name="c", subcore_axis_name="s")
    nw = mesh.num_cores * mesh.num_subcores
    assert bh % nw == 0 and s % L == 0
    rows_per = total_rows // nw
    bh_per = bh // nw
    n_idx = bh_per * s

    pos = input_pos.astype(jnp.int32)
    kval3 = k_val.reshape(bh, s, d)
    vval2 = v_val.reshape(bh * s, d)
    kcache3 = k_cache.reshape(bh, s_max, d)
    vcache2 = v_cache.reshape(total_rows, d)

    bh_blk = 4
    grid_spec = pltpu.PrefetchScalarGridSpec(
        num_scalar_prefetch=1,
        grid=(bh // bh_blk,),
        in_specs=[
            pl.BlockSpec((bh_blk, s_max, d), lambda i, pos_ref: (i, 0, 0)),
            pl.BlockSpec((bh_blk, s, d), lambda i, pos_ref: (i, 0, 0)),
        ],
        out_specs=pl.BlockSpec((bh_blk, s_max, d),
                               lambda i, pos_ref: (i, 0, 0)),
    )
    k_out = pl.pallas_call(
        functools.partial(_tc_k_update, bh_blk=bh_blk, s=s),
        grid_spec=grid_spec,
        out_shape=jax.ShapeDtypeStruct((bh, s_max, d), k_cache.dtype),
    )(pos, kcache3, kval3)

    fn = pl.kernel(
        functools.partial(_sc_v_update, nw=nw, rows_per=rows_per,
                          bh_per=bh_per, s_max=s_max, s=s),
        out_type=jax.ShapeDtypeStruct((total_rows, d), v_cache.dtype),
        mesh=mesh,
        scratch_types=[
            pltpu.VMEM((s + L,), jnp.int32),
            pltpu.VMEM((s + L,), jnp.int32),
            pltpu.VMEM((n_idx,), jnp.int32),
            pltpu.VMEM((n_idx,), jnp.int32),
            pltpu.VMEM((n_idx, d), jnp.float32),
            [pltpu.VMEM((128, d), jnp.float32) for _ in range(4)],
            [pltpu.SemaphoreType.DMA for _ in range(4)],
            [pltpu.SemaphoreType.DMA for _ in range(4)],
            pltpu.SemaphoreType.DMA,
            pltpu.SemaphoreType.DMA,
        ],
    )
    v_out = fn(pos, vval2, vcache2)
    return (k_out.reshape(b, h, s_max, d), v_out.reshape(b, h, s_max, d))
